# Initial kernel scaffold; baseline (speedup 1.0000x reference)
import functools

import jax
import jax.numpy as jnp
from jax import lax
from jax.experimental import pallas as pl
from jax.experimental.pallas import tpu as pltpu
from jax.experimental.pallas import tpu_sc as plsc

VOCAB = 1000000
EMBED = 32
BATCH = 16384
HIST = 50

B = BATCH * HIST
NC = 2
NS = 16
NW = NC * NS
PER_W = B // NW
SUB_LEN = 128
SUB = 8
CHUNK = SUB * SUB_LEN
CHUNKS = PER_W // CHUNK

_mesh = plsc.VectorSubcoreMesh(core_axis_name="c", subcore_axis_name="s")


@functools.partial(
    pl.kernel,
    out_type=jax.ShapeDtypeStruct((B, EMBED), jnp.float32),
    mesh=_mesh,
    scratch_types=[
        pltpu.VMEM((SUB, SUB_LEN), jnp.int32),
        pltpu.VMEM((CHUNK, EMBED), jnp.float32),
        pltpu.SemaphoreType.DMA,
    ],
)
def _emb_lookup(idx_hbm, table_hbm, out_hbm, idx_v, rows_v, sem):
    wid = lax.axis_index("s") * NC + lax.axis_index("c")
    idx_row_base = wid * (PER_W // SUB_LEN)
    out_base = wid * PER_W

    def body(c, carry):
        pltpu.sync_copy(idx_hbm.at[pl.ds(idx_row_base + c * SUB, SUB)], idx_v)
        copies = [
            pltpu.async_copy(
                table_hbm.at[idx_v.at[j]],
                rows_v.at[pl.ds(j * SUB_LEN, SUB_LEN)],
                sem,
            )
            for j in range(SUB)
        ]
        for cp in copies:
            cp.wait()
        pltpu.sync_copy(rows_v, out_hbm.at[pl.ds(out_base + c * CHUNK, CHUNK)])
        return carry

    lax.fori_loop(0, CHUNKS, body, 0)


def kernel(x, table):
    idx = x.reshape(B // SUB_LEN, SUB_LEN)
    out = _emb_lookup(idx, table)
    return out.reshape(BATCH, HIST, EMBED)


# speedup vs baseline: 1.0950x; 1.0950x over previous
import functools

import jax
import jax.numpy as jnp
from jax import lax
from jax.experimental import pallas as pl
from jax.experimental.pallas import tpu as pltpu
from jax.experimental.pallas import tpu_sc as plsc

VOCAB = 1000000
EMBED = 32
BATCH = 16384
HIST = 50

B = BATCH * HIST
NC = 2
NS = 16
NW = NC * NS
PER_W = B // NW
SUB_LEN = 128
SUB = 8
CHUNK = SUB * SUB_LEN
CHUNKS = PER_W // CHUNK

@functools.cache
def _build():
    mesh = plsc.VectorSubcoreMesh(core_axis_name="c", subcore_axis_name="s")

    @functools.partial(
        pl.kernel,
        out_type=jax.ShapeDtypeStruct((B, EMBED), jnp.float32),
        mesh=mesh,
        scratch_types=[
            pltpu.VMEM((SUB, SUB_LEN), jnp.int32),
            pltpu.VMEM((CHUNK, EMBED), jnp.float32),
            pltpu.SemaphoreType.DMA,
        ],
        compiler_params=pltpu.CompilerParams(use_tc_tiling_on_sc=False),
    )
    def emb_lookup(idx_hbm, table_hbm, out_hbm, idx_v, rows_v, sem):
        wid = lax.axis_index("s") * NC + lax.axis_index("c")
        idx_row_base = wid * (PER_W // SUB_LEN)
        out_base = wid * PER_W

        def body(c, carry):
            pltpu.sync_copy(
                idx_hbm.at[pl.ds(idx_row_base + c * SUB, SUB)], idx_v)
            copies = [
                pltpu.async_copy(
                    table_hbm.at[idx_v.at[j]],
                    rows_v.at[pl.ds(j * SUB_LEN, SUB_LEN)],
                    sem,
                )
                for j in range(SUB)
            ]
            for cp in copies:
                cp.wait()
            pltpu.sync_copy(
                rows_v, out_hbm.at[pl.ds(out_base + c * CHUNK, CHUNK)])
            return carry

        lax.fori_loop(0, CHUNKS, body, 0)

    return emb_lookup


def kernel(x, table):
    idx = x.reshape(B // SUB_LEN, SUB_LEN)
    out = _build()(idx, table)
    return out.reshape(BATCH, HIST, EMBED)


# speedup vs baseline: 1.1141x; 1.0174x over previous
import functools

import jax
import jax.numpy as jnp
from jax import lax
from jax.experimental import pallas as pl
from jax.experimental.pallas import tpu as pltpu
from jax.experimental.pallas import tpu_sc as plsc

VOCAB = 1000000
EMBED = 32
BATCH = 16384
HIST = 50

B = BATCH * HIST
NC = 2
NS = 16
NW = NC * NS
PER_W = B // NW
SUB_LEN = 128
SUB = 5
CHUNK = SUB * SUB_LEN
CHUNKS = PER_W // CHUNK
IDX_ROWS = PER_W // SUB_LEN
NBUF = 4


@functools.cache
def _build():
    mesh = plsc.VectorSubcoreMesh(core_axis_name="c", subcore_axis_name="s")

    @functools.partial(
        pl.kernel,
        out_type=jax.ShapeDtypeStruct((B, EMBED), jnp.float32),
        mesh=mesh,
        scratch_types=[
            pltpu.VMEM((IDX_ROWS, SUB_LEN), jnp.int32),
            pltpu.VMEM((NBUF, CHUNK, EMBED), jnp.float32),
            pltpu.SemaphoreType.DMA,
            pltpu.SemaphoreType.DMA,
            pltpu.SemaphoreType.DMA,
            pltpu.SemaphoreType.DMA,
            pltpu.SemaphoreType.DMA,
            pltpu.SemaphoreType.DMA,
            pltpu.SemaphoreType.DMA,
            pltpu.SemaphoreType.DMA,
        ],
        compiler_params=pltpu.CompilerParams(use_tc_tiling_on_sc=False),
    )
    def emb_lookup(idx_hbm, table_hbm, out_hbm, idx_v, rows_v,
                   g0, g1, g2, g3, s0, s1, s2, s3):
        g = (g0, g1, g2, g3)
        s = (s0, s1, s2, s3)
        wid = lax.axis_index("s") * NC + lax.axis_index("c")
        out_base = wid * PER_W

        pltpu.sync_copy(idx_hbm.at[pl.ds(wid * IDX_ROWS, IDX_ROWS)], idx_v)

        def fire_gathers(c, b):
            for j in range(SUB):
                pltpu.async_copy(
                    table_hbm.at[idx_v.at[c * SUB + j]],
                    rows_v.at[b, pl.ds(j * SUB_LEN, SUB_LEN)],
                    g[b],
                )

        def wait_gathers(b):
            pltpu.make_async_copy(
                table_hbm.at[pl.ds(0, CHUNK)], rows_v.at[b], g[b]).wait()

        def fire_store(c, b):
            pltpu.async_copy(
                rows_v.at[b], out_hbm.at[pl.ds(out_base + c * CHUNK, CHUNK)],
                s[b])

        def wait_store(b):
            pltpu.make_async_copy(
                rows_v.at[b], out_hbm.at[pl.ds(0, CHUNK)], s[b]).wait()

        for b in range(NBUF - 1):
            fire_gathers(b, b)

        wait_gathers(0)
        fire_store(0, 0)
        fire_gathers(NBUF - 1, NBUF - 1)

        def body(i, carry):
            for k in range(NBUF):
                c = NBUF * i + 1 + k
                b = (k + 1) % NBUF
                bn = k % NBUF
                wait_gathers(b)
                fire_store(c, b)
                wait_store(bn)
                fire_gathers(c + NBUF - 1, bn)
            return carry

        lax.fori_loop(0, (CHUNKS - NBUF) // NBUF, body, 0)

        for k in range(NBUF - 1):
            c = CHUNKS - (NBUF - 1) + k
            b = c % NBUF
            wait_gathers(b)
            fire_store(c, b)
        for b in range(NBUF):
            wait_store(b)

    return emb_lookup


def kernel(x, table):
    idx = x.reshape(B // SUB_LEN, SUB_LEN)
    out = _build()(idx, table)
    return out.reshape(BATCH, HIST, EMBED)


# speedup vs baseline: 1.4646x; 1.3146x over previous
import functools

import jax
import jax.numpy as jnp
from jax import lax
from jax.experimental import pallas as pl
from jax.experimental.pallas import tpu as pltpu
from jax.experimental.pallas import tpu_sc as plsc

VOCAB = 1000000
EMBED = 32
BATCH = 16384
HIST = 50

NC = 2
NS = 16
NW = NC * NS
BLK = BATCH // NW
SUB_LEN = 128
SUB = BLK // SUB_LEN
L = 16


@functools.cache
def _build():
    mesh = plsc.VectorSubcoreMesh(core_axis_name="c", subcore_axis_name="s")

    @functools.partial(
        pl.kernel,
        out_type=jax.ShapeDtypeStruct((HIST, EMBED, BATCH), jnp.float32),
        mesh=mesh,
        scratch_types=[
            pltpu.VMEM((2, SUB, SUB_LEN), jnp.int32),
            pltpu.VMEM((2, BLK, EMBED), jnp.float32),
            pltpu.VMEM((2, EMBED, BLK), jnp.float32),
            pltpu.SemaphoreType.DMA,
            pltpu.SemaphoreType.DMA,
            pltpu.SemaphoreType.DMA,
            pltpu.SemaphoreType.DMA,
        ],
        compiler_params=pltpu.CompilerParams(
            use_tc_tiling_on_sc=False, needs_layout_passes=False),
    )
    def emb_lookup(xt_hbm, table_hbm, out_hbm, idx_v, rows_v, outt_v,
                   g0, g1, s0, s1):
        g = (g0, g1)
        s = (s0, s1)
        wid = lax.axis_index("s") * NC + lax.axis_index("c")
        b0 = wid * BLK
        grp0 = wid * SUB

        def fire_gathers(h, b):
            pltpu.sync_copy(xt_hbm.at[h, pl.ds(grp0, SUB)], idx_v.at[b])
            for j in range(SUB):
                pltpu.async_copy(
                    table_hbm.at[idx_v.at[b, j]],
                    rows_v.at[b, pl.ds(j * SUB_LEN, SUB_LEN)],
                    g[b],
                )

        def wait_gathers(b):
            pltpu.make_async_copy(
                table_hbm.at[pl.ds(0, BLK)], rows_v.at[b], g[b]).wait()

        def transpose(b):
            rows = rows_v.at[b]
            outt = outt_v.at[b]

            def col_body(c, carry):
                cidx = jnp.full((L,), c, jnp.int32)
                for j16 in range(BLK // L):
                    ridx = lax.iota(jnp.int32, L) + j16 * L
                    v = plsc.load_gather(rows, [ridx, cidx])
                    outt[c, pl.ds(j16 * L, L)] = v
                return carry

            lax.fori_loop(0, EMBED, col_body, 0)

        def fire_store(h, b):
            pltpu.async_copy(
                outt_v.at[b], out_hbm.at[h, :, pl.ds(b0, BLK)], s[b])

        def wait_store(b):
            pltpu.make_async_copy(
                outt_v.at[b], out_hbm.at[0, :, pl.ds(0, BLK)], s[b]).wait()

        fire_gathers(0, 0)
        fire_gathers(1, 1)
        wait_gathers(0)
        transpose(0)
        fire_store(0, 0)
        fire_gathers(2, 0)
        wait_gathers(1)
        transpose(1)
        fire_store(1, 1)

        def body(i, carry):
            for k in range(2):
                h = 2 * i + 2 + k
                b = k
                fire_gathers(h + 1, 1 - b)
                wait_gathers(b)
                wait_store(b)
                transpose(b)
                fire_store(h, b)
            return carry

        lax.fori_loop(0, (HIST - 4) // 2, body, 0)

        fire_gathers(HIST - 1, (HIST - 1) % 2)
        for h in (HIST - 2, HIST - 1):
            b = h % 2
            wait_gathers(b)
            wait_store(b)
            transpose(b)
            fire_store(h, b)
        wait_store(0)
        wait_store(1)

    return emb_lookup


def kernel(x, table):
    xt = x.T.reshape(HIST, BATCH // SUB_LEN, SUB_LEN)
    out = _build()(xt, table)
    return out.transpose(2, 0, 1)


# speedup vs baseline: 1.4658x; 1.0008x over previous
import functools

import jax
import jax.numpy as jnp
from jax import lax
from jax.experimental import pallas as pl
from jax.experimental.pallas import tpu as pltpu
from jax.experimental.pallas import tpu_sc as plsc

VOCAB = 1000000
EMBED = 32
BATCH = 16384
HIST = 50

NC = 2
NS = 16
NW = NC * NS
BLK = BATCH // NW
SUB_LEN = 128
SUB = BLK // SUB_LEN
L = 16


@functools.cache
def _build():
    mesh = plsc.VectorSubcoreMesh(core_axis_name="c", subcore_axis_name="s")

    @functools.partial(
        pl.kernel,
        out_type=jax.ShapeDtypeStruct((HIST, EMBED, BATCH), jnp.float32),
        mesh=mesh,
        scratch_types=[
            pltpu.VMEM((2, SUB, SUB_LEN), jnp.int32),
            pltpu.VMEM((2, BLK, EMBED), jnp.float32),
            pltpu.VMEM((2, EMBED, BLK), jnp.float32),
            pltpu.SemaphoreType.DMA,
            pltpu.SemaphoreType.DMA,
            pltpu.SemaphoreType.DMA,
            pltpu.SemaphoreType.DMA,
        ],
        compiler_params=pltpu.CompilerParams(
            use_tc_tiling_on_sc=False, needs_layout_passes=False),
    )
    def emb_lookup(xt_hbm, table_hbm, out_hbm, idx_v, rows_v, outt_v,
                   g0, g1, s0, s1):
        g = (g0, g1)
        s = (s0, s1)
        wid = lax.axis_index("s") * NC + lax.axis_index("c")
        b0 = wid * BLK
        grp0 = wid * SUB

        def fire_gathers(h, b):
            pltpu.sync_copy(xt_hbm.at[h, pl.ds(grp0, SUB)], idx_v.at[b])
            for j in range(SUB):
                pltpu.async_copy(
                    table_hbm.at[idx_v.at[b, j]],
                    rows_v.at[b, pl.ds(j * SUB_LEN, SUB_LEN)],
                    g[b],
                )

        def wait_gathers(b):
            pltpu.make_async_copy(
                table_hbm.at[pl.ds(0, BLK)], rows_v.at[b], g[b]).wait()

        def transpose(b):
            rows = rows_v.at[b]
            outt = outt_v.at[b]

            def j_body(j16, carry):
                base = j16 * L
                ridx = lax.iota(jnp.int32, L) + base
                for c in range(EMBED):
                    cidx = jnp.full((L,), c, jnp.int32)
                    v = plsc.load_gather(rows, [ridx, cidx])
                    outt[c, pl.ds(base, L)] = v
                return carry

            lax.fori_loop(0, BLK // L, j_body, 0)

        def fire_store(h, b):
            pltpu.async_copy(
                outt_v.at[b], out_hbm.at[h, :, pl.ds(b0, BLK)], s[b])

        def wait_store(b):
            pltpu.make_async_copy(
                outt_v.at[b], out_hbm.at[0, :, pl.ds(0, BLK)], s[b]).wait()

        fire_gathers(0, 0)
        fire_gathers(1, 1)
        wait_gathers(0)
        transpose(0)
        fire_store(0, 0)
        fire_gathers(2, 0)
        wait_gathers(1)
        transpose(1)
        fire_store(1, 1)

        def body(i, carry):
            for k in range(2):
                h = 2 * i + 2 + k
                b = k
                fire_gathers(h + 1, 1 - b)
                wait_gathers(b)
                wait_store(b)
                transpose(b)
                fire_store(h, b)
            return carry

        lax.fori_loop(0, (HIST - 4) // 2, body, 0)

        fire_gathers(HIST - 1, (HIST - 1) % 2)
        for h in (HIST - 2, HIST - 1):
            b = h % 2
            wait_gathers(b)
            wait_store(b)
            transpose(b)
            fire_store(h, b)
        wait_store(0)
        wait_store(1)

    return emb_lookup


def kernel(x, table):
    xt = x.T.reshape(HIST, BATCH // SUB_LEN, SUB_LEN)
    out = _build()(xt, table)
    return out.transpose(2, 0, 1)


# speedup vs baseline: 2.1968x; 1.4987x over previous
import functools

import jax
import jax.numpy as jnp
from jax import lax
from jax.experimental import pallas as pl
from jax.experimental.pallas import tpu as pltpu
from jax.experimental.pallas import tpu_sc as plsc

VOCAB = 1000000
EMBED = 32
BATCH = 16384
HIST = 50

NC = 2
NS = 16
NW = NC * NS
BLK = BATCH // NW
SUB_LEN = 128
SUB = BLK // SUB_LEN
L = 16


@functools.cache
def _build():
    mesh = plsc.VectorSubcoreMesh(core_axis_name="c", subcore_axis_name="s")

    @functools.partial(
        pl.kernel,
        out_type=jax.ShapeDtypeStruct((HIST, EMBED, BATCH), jnp.float32),
        mesh=mesh,
        scratch_types=[
            pltpu.VMEM((2, SUB, SUB_LEN), jnp.int32),
            pltpu.VMEM((2, BLK, EMBED), jnp.float32),
            pltpu.VMEM((2, EMBED, BLK), jnp.float32),
            pltpu.SemaphoreType.DMA,
            pltpu.SemaphoreType.DMA,
            pltpu.SemaphoreType.DMA,
            pltpu.SemaphoreType.DMA,
        ],
        compiler_params=pltpu.CompilerParams(
            use_tc_tiling_on_sc=False, needs_layout_passes=False),
    )
    def emb_lookup(xt_hbm, table_hbm, out_hbm, idx_v, rows_v, outt_v,
                   g0, g1, s0, s1):
        g = (g0, g1)
        s = (s0, s1)
        wid = lax.axis_index("s") * NC + lax.axis_index("c")
        b0 = wid * BLK
        grp0 = wid * SUB

        def fire_gathers(h, b):
            pltpu.sync_copy(xt_hbm.at[h, pl.ds(grp0, SUB)], idx_v.at[b])
            for j in range(SUB):
                pltpu.async_copy(
                    table_hbm.at[idx_v.at[b, j]],
                    rows_v.at[b, pl.ds(j * SUB_LEN, SUB_LEN)],
                    g[b],
                )

        def wait_gathers(b):
            pltpu.make_async_copy(
                table_hbm.at[pl.ds(0, BLK)], rows_v.at[b], g[b]).wait()

        def transpose(b):
            rows = rows_v.at[b]
            outt = outt_v.at[b]
            lanes = lax.iota(jnp.int32, L)

            def j_body(j16, carry):
                jidx = lanes + j16 * L
                for c0 in range(0, EMBED, L):
                    for k in range(L):
                        cd = c0 + ((lanes + k) % L)
                        v = plsc.load_gather(rows, [jidx, cd])
                        plsc.store_scatter(outt, [cd, jidx], v)
                return carry

            lax.fori_loop(0, BLK // L, j_body, 0)

        def fire_store(h, b):
            pltpu.async_copy(
                outt_v.at[b], out_hbm.at[h, :, pl.ds(b0, BLK)], s[b])

        def wait_store(b):
            pltpu.make_async_copy(
                outt_v.at[b], out_hbm.at[0, :, pl.ds(0, BLK)], s[b]).wait()

        fire_gathers(0, 0)
        fire_gathers(1, 1)
        wait_gathers(0)
        transpose(0)
        fire_store(0, 0)
        fire_gathers(2, 0)
        wait_gathers(1)
        transpose(1)
        fire_store(1, 1)

        def body(i, carry):
            for k in range(2):
                h = 2 * i + 2 + k
                b = k
                fire_gathers(h + 1, 1 - b)
                wait_gathers(b)
                wait_store(b)
                transpose(b)
                fire_store(h, b)
            return carry

        lax.fori_loop(0, (HIST - 4) // 2, body, 0)

        fire_gathers(HIST - 1, (HIST - 1) % 2)
        for h in (HIST - 2, HIST - 1):
            b = h % 2
            wait_gathers(b)
            wait_store(b)
            transpose(b)
            fire_store(h, b)
        wait_store(0)
        wait_store(1)

    return emb_lookup


def kernel(x, table):
    xt = x.T.reshape(HIST, BATCH // SUB_LEN, SUB_LEN)
    out = _build()(xt, table)
    return out.transpose(2, 0, 1)


# speedup vs baseline: 2.2831x; 1.0393x over previous
import functools

import jax
import jax.numpy as jnp
from jax import lax
from jax.experimental import pallas as pl
from jax.experimental.pallas import tpu as pltpu
from jax.experimental.pallas import tpu_sc as plsc

VOCAB = 1000000
EMBED = 32
BATCH = 16384
HIST = 50

NC = 2
NS = 16
NW = NC * NS
BLK = BATCH // NW
SUB_LEN = 128
SUB = BLK // SUB_LEN
L = 16


@functools.cache
def _build():
    mesh = plsc.VectorSubcoreMesh(core_axis_name="c", subcore_axis_name="s")

    @functools.partial(
        pl.kernel,
        out_type=jax.ShapeDtypeStruct((HIST, EMBED, BATCH), jnp.float32),
        mesh=mesh,
        scratch_types=[
            pltpu.VMEM((HIST, SUB, SUB_LEN), jnp.int32),
            pltpu.VMEM((2, BLK, EMBED), jnp.float32),
            pltpu.VMEM((2, EMBED, BLK), jnp.float32),
            pltpu.SemaphoreType.DMA,
            pltpu.SemaphoreType.DMA,
            pltpu.SemaphoreType.DMA,
            pltpu.SemaphoreType.DMA,
        ],
        compiler_params=pltpu.CompilerParams(
            use_tc_tiling_on_sc=False, needs_layout_passes=False),
    )
    def emb_lookup(xt_hbm, table_hbm, out_hbm, idx_v, rows_v, outt_v,
                   g0, g1, s0, s1):
        g = (g0, g1)
        s = (s0, s1)
        wid = lax.axis_index("s") * NC + lax.axis_index("c")
        b0 = wid * BLK
        grp0 = wid * SUB

        pltpu.sync_copy(xt_hbm.at[:, pl.ds(grp0, SUB)], idx_v)

        def fire_gathers(h, b):
            for j in range(SUB):
                pltpu.async_copy(
                    table_hbm.at[idx_v.at[h, j]],
                    rows_v.at[b, pl.ds(j * SUB_LEN, SUB_LEN)],
                    g[b],
                )

        def wait_gathers(b):
            pltpu.make_async_copy(
                table_hbm.at[pl.ds(0, BLK)], rows_v.at[b], g[b]).wait()

        def transpose(b):
            rows = rows_v.at[b]
            outt = outt_v.at[b]
            lanes = lax.iota(jnp.int32, L)

            def j_body(j16, carry):
                jidx = lanes + j16 * L
                for c0 in range(0, EMBED, L):
                    for k in range(L):
                        cd = c0 + ((lanes + k) % L)
                        v = plsc.load_gather(rows, [jidx, cd])
                        plsc.store_scatter(outt, [cd, jidx], v)
                return carry

            lax.fori_loop(0, BLK // L, j_body, 0)

        def fire_store(h, b):
            pltpu.async_copy(
                outt_v.at[b], out_hbm.at[h, :, pl.ds(b0, BLK)], s[b])

        def wait_store(b):
            pltpu.make_async_copy(
                outt_v.at[b], out_hbm.at[0, :, pl.ds(0, BLK)], s[b]).wait()

        fire_gathers(0, 0)
        fire_gathers(1, 1)
        wait_gathers(0)
        transpose(0)
        fire_store(0, 0)
        fire_gathers(2, 0)
        wait_gathers(1)
        transpose(1)
        fire_store(1, 1)

        def body(i, carry):
            for k in range(2):
                h = 2 * i + 2 + k
                b = k
                fire_gathers(h + 1, 1 - b)
                wait_gathers(b)
                wait_store(b)
                transpose(b)
                fire_store(h, b)
            return carry

        lax.fori_loop(0, (HIST - 4) // 2, body, 0)

        fire_gathers(HIST - 1, (HIST - 1) % 2)
        for h in (HIST - 2, HIST - 1):
            b = h % 2
            wait_gathers(b)
            wait_store(b)
            transpose(b)
            fire_store(h, b)
        wait_store(0)
        wait_store(1)

    return emb_lookup


def kernel(x, table):
    xt = x.T.reshape(HIST, BATCH // SUB_LEN, SUB_LEN)
    out = _build()(xt, table)
    return out.transpose(2, 0, 1)
